# baseline (speedup 1.0000x reference)
import functools

import jax
import jax.numpy as jnp
from jax.experimental import pallas as pl
from jax.experimental.pallas import tpu as pltpu

N = 134400
C = 64
W = 16
L = 128
N2 = N // 2
R = 4200
NB = N2 // R


def _body(stu_ref, tea_ref, se_ref, so_ref, te_ref, to_ref,
          sge_ref, sgo_ref, tge_ref, tgo_ref, me_ref, mo_ref,
          out_ref, acc_ref):
    pid = pl.program_id(0)

    @pl.when(pid == 0)
    def _init():
        acc_ref[0] = 0.0
        acc_ref[1] = 0.0

    mask_e = jnp.logical_and(te_ref[...] >= se_ref[...],
                             tge_ref[...] == sge_ref[...])
    mask_e = jnp.logical_and(mask_e, me_ref[...] != 0).astype(jnp.float32)
    mask_o = jnp.logical_and(to_ref[...] >= so_ref[...],
                             tgo_ref[...] == sgo_ref[...])
    mask_o = jnp.logical_and(mask_o, mo_ref[...] != 0).astype(jnp.float32)

    t = tea_ref[...]
    s = stu_ref[...]

    gi = jax.lax.broadcasted_iota(jnp.int32, (L, L), 0) // W
    gj = jax.lax.broadcasted_iota(jnp.int32, (L, L), 1) // W
    p_mat = (gi == gj).astype(jnp.float32)

    et = jnp.exp(t)
    es = jnp.exp(s)
    bt = jax.lax.dot(et, p_mat)
    bs = jax.lax.dot(es, p_mat)

    elem = (et / bt) * (t - s) - jnp.log(bt / bs) * (1.0 / W)

    lane = jax.lax.broadcasted_iota(jnp.int32, (R, L), 1)
    w = jnp.where(lane < C, mask_e, mask_o)

    acc_ref[0] += jnp.sum(elem * w)
    acc_ref[1] += jnp.sum(mask_e) + jnp.sum(mask_o)

    @pl.when(pid == NB - 1)
    def _fin():
        out_ref[0, 0] = acc_ref[0] / (jnp.maximum(acc_ref[1], 1.0) * C)


@functools.partial(jax.jit, static_argnames=())
def kernel(stu_distri, tea_distri, stu_candidate_iou, tea_candidate_iou,
           stu_target_gt_idx, tea_target_gt_idx, Ms):
    sd = stu_distri.reshape(N2, L)
    td = tea_distri.reshape(N2, L)

    def split(x, dtype=None):
        x2 = x.reshape(N2, 2)
        if dtype is not None:
            x2 = x2.astype(dtype)
        return x2[:, 0:1], x2[:, 1:2]

    se, so = split(stu_candidate_iou)
    te, to = split(tea_candidate_iou)
    sge, sgo = split(stu_target_gt_idx, jnp.int32)
    tge, tgo = split(tea_target_gt_idx, jnp.int32)
    me, mo = split(Ms, jnp.int32)

    row_spec = pl.BlockSpec((R, L), lambda i: (i, 0))
    vec_spec = pl.BlockSpec((R, 1), lambda i: (i, 0))

    out = pl.pallas_call(
        _body,
        grid=(NB,),
        in_specs=[row_spec, row_spec] + [vec_spec] * 10,
        out_specs=pl.BlockSpec(memory_space=pltpu.SMEM),
        out_shape=jax.ShapeDtypeStruct((1, 1), jnp.float32),
        scratch_shapes=[pltpu.SMEM((2,), jnp.float32)],
        compiler_params=pltpu.CompilerParams(
            dimension_semantics=("arbitrary",)),
    )(sd, td, se, so, te, to, sge, sgo, tge, tgo, me, mo)
    return out[0, 0]


# speedup vs baseline: 1.2424x; 1.2424x over previous
import functools

import jax
import jax.numpy as jnp
from jax.experimental import pallas as pl
from jax.experimental.pallas import tpu as pltpu

N = 134400
C = 64
W = 16
L = 128
N2 = N // 2
R = 4200
NB = N2 // R


def _mask(t_iou, s_iou, t_gt, s_gt, ms):
    m = jnp.logical_and(t_iou >= s_iou, t_gt == s_gt)
    return jnp.logical_and(m, ms != 0).astype(jnp.float32)


def _body(stu_ref, tea_ref, se_ref, so_ref, te_ref, to_ref,
          sge_ref, sgo_ref, tge_ref, tgo_ref, me_ref, mo_ref,
          out_ref, acc_ref):
    pid = pl.program_id(0)

    @pl.when(pid == 0)
    def _init():
        acc_ref[0] = 0.0
        acc_ref[1] = 0.0

    m_e = _mask(te_ref[0, 0, :], se_ref[0, 0, :], tge_ref[0, 0, :],
                sge_ref[0, 0, :], me_ref[0, 0, :])
    m_o = _mask(to_ref[0, 0, :], so_ref[0, 0, :], tgo_ref[0, 0, :],
                sgo_ref[0, 0, :], mo_ref[0, 0, :])

    t = tea_ref[...]
    s = stu_ref[...]

    gi = jax.lax.broadcasted_iota(jnp.int32, (L, L), 0) // W
    gj = jax.lax.broadcasted_iota(jnp.int32, (L, L), 1) // W
    p_mat = (gi == gj).astype(jnp.float32)

    et = jnp.exp(t)
    es = jnp.exp(s)
    bt = jax.lax.dot(et, p_mat)
    bs = jax.lax.dot(es, p_mat)

    elem = (et / bt) * (t - s) - jnp.log(bt / bs) * (1.0 / W)

    m2 = jnp.concatenate([m_e.reshape(1, R), m_o.reshape(1, R)], axis=0)
    g = jax.lax.dot(m2, elem)

    lane = jax.lax.broadcasted_iota(jnp.int32, (2, L), 1)
    half = jax.lax.broadcasted_iota(jnp.int32, (2, L), 0)
    keep = jnp.where((lane < C) == (half == 0), 1.0, 0.0)

    acc_ref[0] += jnp.sum(g * keep)
    acc_ref[1] += jnp.sum(m_e) + jnp.sum(m_o)

    @pl.when(pid == NB - 1)
    def _fin():
        out_ref[0, 0] = acc_ref[0] / (jnp.maximum(acc_ref[1], 1.0) * C)


@functools.partial(jax.jit, static_argnames=())
def kernel(stu_distri, tea_distri, stu_candidate_iou, tea_candidate_iou,
           stu_target_gt_idx, tea_target_gt_idx, Ms):
    sd = stu_distri.reshape(N2, L)
    td = tea_distri.reshape(N2, L)

    def split(x, dtype=None):
        x2 = x.reshape(N2, 2)
        if dtype is not None:
            x2 = x2.astype(dtype)
        return (x2[:, 0].reshape(NB, 1, R), x2[:, 1].reshape(NB, 1, R))

    se, so = split(stu_candidate_iou)
    te, to = split(tea_candidate_iou)
    sge, sgo = split(stu_target_gt_idx.astype(jnp.int32))
    tge, tgo = split(tea_target_gt_idx.astype(jnp.int32))
    me, mo = split(Ms.astype(jnp.int32))

    row_spec = pl.BlockSpec((R, L), lambda i: (i, 0))
    vec_spec = pl.BlockSpec((1, 1, R), lambda i: (i, 0, 0))

    out = pl.pallas_call(
        _body,
        grid=(NB,),
        in_specs=[row_spec, row_spec] + [vec_spec] * 10,
        out_specs=pl.BlockSpec(memory_space=pltpu.SMEM),
        out_shape=jax.ShapeDtypeStruct((1, 1), jnp.float32),
        scratch_shapes=[pltpu.SMEM((2,), jnp.float32)],
        compiler_params=pltpu.CompilerParams(
            dimension_semantics=("arbitrary",)),
    )(sd, td, se, so, te, to, sge, sgo, tge, tgo, me, mo)
    return out[0, 0]


# speedup vs baseline: 4.0158x; 3.2323x over previous
import functools

import jax
import jax.numpy as jnp
from jax.experimental import pallas as pl
from jax.experimental.pallas import tpu as pltpu

N = 134400
C = 64
W = 16
R = 4200
NB = N // R


def _body(stu_ref, tea_ref, siou_ref, tiou_ref, sgt_ref, tgt_ref, ms_ref,
          out_ref, acc_ref):
    pid = pl.program_id(0)

    @pl.when(pid == 0)
    def _init():
        acc_ref[0] = 0.0
        acc_ref[1] = 0.0

    m = jnp.logical_and(tiou_ref[0, 0, :] >= siou_ref[0, 0, :],
                        tgt_ref[0, 0, :] == sgt_ref[0, 0, :])
    m = jnp.logical_and(m, ms_ref[0, 0, :] != 0).astype(jnp.float32)

    t = tea_ref[...]
    s = stu_ref[...]

    gi = jax.lax.broadcasted_iota(jnp.int32, (C, C), 0) // W
    gj = jax.lax.broadcasted_iota(jnp.int32, (C, C), 1) // W
    p_mat = (gi == gj).astype(jnp.float32)

    et = jnp.exp(t)
    es = jnp.exp(s)
    bt = jax.lax.dot(et, p_mat)
    bs = jax.lax.dot(es, p_mat)

    elem = (et / bt) * (t - s) - jnp.log(bt / bs) * (1.0 / W)

    g = jax.lax.dot(m.reshape(1, R), elem)

    acc_ref[0] += jnp.sum(g)
    acc_ref[1] += jnp.sum(m)

    @pl.when(pid == NB - 1)
    def _fin():
        out_ref[0, 0] = acc_ref[0] / (jnp.maximum(acc_ref[1], 1.0) * C)


@functools.partial(jax.jit, static_argnames=())
def kernel(stu_distri, tea_distri, stu_candidate_iou, tea_candidate_iou,
           stu_target_gt_idx, tea_target_gt_idx, Ms):
    siou = stu_candidate_iou.reshape(NB, 1, R)
    tiou = tea_candidate_iou.reshape(NB, 1, R)
    sgt = stu_target_gt_idx.reshape(NB, 1, R)
    tgt = tea_target_gt_idx.reshape(NB, 1, R)
    if sgt.dtype != jnp.int32:
        sgt = sgt.astype(jnp.int32)
        tgt = tgt.astype(jnp.int32)
    ms = Ms.reshape(NB, 1, R)

    row_spec = pl.BlockSpec((R, C), lambda i: (i, 0))
    vec_spec = pl.BlockSpec((1, 1, R), lambda i: (i, 0, 0))

    out = pl.pallas_call(
        _body,
        grid=(NB,),
        in_specs=[row_spec, row_spec] + [vec_spec] * 5,
        out_specs=pl.BlockSpec(memory_space=pltpu.SMEM),
        out_shape=jax.ShapeDtypeStruct((1, 1), jnp.float32),
        scratch_shapes=[pltpu.SMEM((2,), jnp.float32)],
        compiler_params=pltpu.CompilerParams(
            dimension_semantics=("arbitrary",)),
    )(stu_distri, tea_distri, siou, tiou, sgt, tgt, ms)
    return out[0, 0]


# speedup vs baseline: 4.3308x; 1.0784x over previous
import functools

import jax
import jax.numpy as jnp
from jax.experimental import pallas as pl
from jax.experimental.pallas import tpu as pltpu

N = 134400
C = 64
W = 16
R = 4096
NB = (N + R - 1) // R
NR = N // 128
RS = R // 128


def _body(stu_ref, tea_ref, siou_ref, tiou_ref, sgt_ref, tgt_ref, ms_ref,
          out_ref, acc_ref):
    pid = pl.program_id(0)

    @pl.when(pid == 0)
    def _init():
        acc_ref[0] = 0.0
        acc_ref[1] = 0.0

    m = jnp.logical_and(tiou_ref[...] >= siou_ref[...],
                        tgt_ref[...] == sgt_ref[...])
    m = jnp.logical_and(m, ms_ref[...])
    srow = jax.lax.broadcasted_iota(jnp.int32, (RS, 128), 0)
    m = jnp.logical_and(m, srow < (NR - pid * RS))
    mf = m.astype(jnp.float32)

    row = jax.lax.broadcasted_iota(jnp.int32, (R, C), 0)
    valid = row < (N - pid * R)
    t = jnp.where(valid, tea_ref[...], 0.0)
    s = jnp.where(valid, stu_ref[...], 0.0)

    gi = jax.lax.broadcasted_iota(jnp.int32, (C, C), 0) // W
    gj = jax.lax.broadcasted_iota(jnp.int32, (C, C), 1) // W
    p_mat = (gi == gj).astype(jnp.float32)

    et = jnp.exp(t)
    es = jnp.exp(s)
    bt = jax.lax.dot(et, p_mat)
    bs = jax.lax.dot(es, p_mat)

    elem = (et / bt) * (t - s) - jnp.log(bt / bs) * (1.0 / W)

    g = jax.lax.dot(mf.reshape(1, R), elem)

    acc_ref[0] += jnp.sum(g)
    acc_ref[1] += jnp.sum(mf)

    @pl.when(pid == NB - 1)
    def _fin():
        out_ref[0, 0] = acc_ref[0] / (jnp.maximum(acc_ref[1], 1.0) * C)


@functools.partial(jax.jit, static_argnames=())
def kernel(stu_distri, tea_distri, stu_candidate_iou, tea_candidate_iou,
           stu_target_gt_idx, tea_target_gt_idx, Ms):
    siou = stu_candidate_iou.reshape(NR, 128)
    tiou = tea_candidate_iou.reshape(NR, 128)
    sgt = stu_target_gt_idx.reshape(NR, 128)
    tgt = tea_target_gt_idx.reshape(NR, 128)
    if sgt.dtype != jnp.int32:
        sgt = sgt.astype(jnp.int32)
        tgt = tgt.astype(jnp.int32)
    ms = Ms.reshape(NR, 128)

    row_spec = pl.BlockSpec((R, C), lambda i: (i, 0))
    vec_spec = pl.BlockSpec((RS, 128), lambda i: (i, 0))

    out = pl.pallas_call(
        _body,
        grid=(NB,),
        in_specs=[row_spec, row_spec] + [vec_spec] * 5,
        out_specs=pl.BlockSpec(memory_space=pltpu.SMEM),
        out_shape=jax.ShapeDtypeStruct((1, 1), jnp.float32),
        scratch_shapes=[pltpu.SMEM((2,), jnp.float32)],
        compiler_params=pltpu.CompilerParams(
            dimension_semantics=("arbitrary",)),
    )(stu_distri, tea_distri, siou, tiou, sgt, tgt, ms)
    return out[0, 0]


# speedup vs baseline: 14.8252x; 3.4232x over previous
import functools

import jax
import jax.numpy as jnp
from jax.experimental import pallas as pl
from jax.experimental.pallas import tpu as pltpu

N = 134400
C = 64
W = 16
RT = 4096
NB = (N + RT - 1) // RT


def _body(stu_ref, tea_ref, siou_ref, tiou_ref, sgt_ref, tgt_ref, ms_ref,
          out_ref, acc_ref):
    pid = pl.program_id(0)

    @pl.when(pid == 0)
    def _init():
        acc_ref[0] = 0.0
        acc_ref[1] = 0.0

    nvalid = N - pid * RT

    lane1 = jax.lax.broadcasted_iota(jnp.int32, (RT,), 0)
    m = jnp.logical_and(tiou_ref[...] >= siou_ref[...],
                        tgt_ref[...] == sgt_ref[...])
    m = jnp.logical_and(m, ms_ref[...])
    m = jnp.logical_and(m, lane1 < nvalid)
    mf = m.astype(jnp.float32)

    lane2 = jax.lax.broadcasted_iota(jnp.int32, (C, RT), 1)
    valid = lane2 < nvalid
    t = jnp.where(valid, tea_ref[...], 0.0)
    s = jnp.where(valid, stu_ref[...], 0.0)

    gi = jax.lax.broadcasted_iota(jnp.int32, (C, C), 0) // W
    gj = jax.lax.broadcasted_iota(jnp.int32, (C, C), 1) // W
    p_mat = (gi == gj).astype(jnp.float32)

    et = jnp.exp(t)
    es = jnp.exp(s)
    bt = jax.lax.dot(p_mat, et)
    bs = jax.lax.dot(p_mat, es)

    elem = (et / bt) * (t - s) - jnp.log(bt / bs) * (1.0 / W)

    row_kl = jnp.sum(elem, axis=0)

    acc_ref[0] += jnp.sum(row_kl * mf)
    acc_ref[1] += jnp.sum(mf)

    @pl.when(pid == NB - 1)
    def _fin():
        out_ref[0, 0] = acc_ref[0] / (jnp.maximum(acc_ref[1], 1.0) * C)


@functools.partial(jax.jit, static_argnames=())
def kernel(stu_distri, tea_distri, stu_candidate_iou, tea_candidate_iou,
           stu_target_gt_idx, tea_target_gt_idx, Ms):
    st = stu_distri.T
    tt = tea_distri.T
    sgt = stu_target_gt_idx
    tgt = tea_target_gt_idx
    if sgt.dtype != jnp.int32:
        sgt = sgt.astype(jnp.int32)
        tgt = tgt.astype(jnp.int32)

    row_spec = pl.BlockSpec((C, RT), lambda i: (0, i))
    vec_spec = pl.BlockSpec((RT,), lambda i: (i,))

    out = pl.pallas_call(
        _body,
        grid=(NB,),
        in_specs=[row_spec, row_spec] + [vec_spec] * 5,
        out_specs=pl.BlockSpec(memory_space=pltpu.SMEM),
        out_shape=jax.ShapeDtypeStruct((1, 1), jnp.float32),
        scratch_shapes=[pltpu.SMEM((2,), jnp.float32)],
        compiler_params=pltpu.CompilerParams(
            dimension_semantics=("arbitrary",)),
    )(st, tt, stu_candidate_iou, tea_candidate_iou, sgt, tgt, Ms)
    return out[0, 0]


# speedup vs baseline: 16.2239x; 1.0943x over previous
import functools

import jax
import jax.numpy as jnp
from jax.experimental import pallas as pl
from jax.experimental.pallas import tpu as pltpu

N = 134400
C = 64
W = 16
RT = 4096
NB = (N + RT - 1) // RT


def _body(stu_ref, tea_ref, siou_ref, tiou_ref, sgt_ref, tgt_ref, ms_ref,
          out_ref, acc_ref):
    pid = pl.program_id(0)

    @pl.when(pid == 0)
    def _init():
        acc_ref[0] = 0.0
        acc_ref[1] = 0.0

    nvalid = N - pid * RT

    lane1 = jax.lax.broadcasted_iota(jnp.int32, (RT,), 0)
    m = jnp.logical_and(tiou_ref[...] >= siou_ref[...],
                        tgt_ref[...] == sgt_ref[...])
    m = jnp.logical_and(m, ms_ref[...])
    m = jnp.logical_and(m, lane1 < nvalid)
    mf = m.astype(jnp.float32)

    lane2 = jax.lax.broadcasted_iota(jnp.int32, (C, RT), 1)
    valid = lane2 < nvalid
    t = jnp.where(valid, tea_ref[...], 0.0)
    s = jnp.where(valid, stu_ref[...], 0.0)

    a8 = jax.lax.broadcasted_iota(jnp.int32, (8, C), 0) & 3
    c8 = jax.lax.broadcasted_iota(jnp.int32, (8, C), 1) // W
    p8 = (a8 == c8).astype(jnp.float32)
    aq = jax.lax.broadcasted_iota(jnp.int32, (C, 8), 1)
    cq = jax.lax.broadcasted_iota(jnp.int32, (C, 8), 0) // W
    q_mat = (aq == cq).astype(jnp.float32)

    et = jnp.exp(t)
    es = jnp.exp(s)
    bt8 = jax.lax.dot(p8, et)
    bs8 = jax.lax.dot(p8, es)
    btinv = jax.lax.dot(q_mat, 1.0 / bt8)

    elem1 = et * btinv * (t - s)
    l4 = jnp.log(bt8[0:4, :] / bs8[0:4, :])

    row_kl = jnp.sum(elem1, axis=0) - jnp.sum(l4, axis=0)

    acc_ref[0] += jnp.sum(row_kl * mf)
    acc_ref[1] += jnp.sum(mf)

    @pl.when(pid == NB - 1)
    def _fin():
        out_ref[0, 0] = acc_ref[0] / (jnp.maximum(acc_ref[1], 1.0) * C)


@functools.partial(jax.jit, static_argnames=())
def kernel(stu_distri, tea_distri, stu_candidate_iou, tea_candidate_iou,
           stu_target_gt_idx, tea_target_gt_idx, Ms):
    st = stu_distri.T
    tt = tea_distri.T
    sgt = stu_target_gt_idx
    tgt = tea_target_gt_idx
    if sgt.dtype != jnp.int32:
        sgt = sgt.astype(jnp.int32)
        tgt = tgt.astype(jnp.int32)

    row_spec = pl.BlockSpec((C, RT), lambda i: (0, i))
    vec_spec = pl.BlockSpec((RT,), lambda i: (i,))

    out = pl.pallas_call(
        _body,
        grid=(NB,),
        in_specs=[row_spec, row_spec] + [vec_spec] * 5,
        out_specs=pl.BlockSpec(memory_space=pltpu.SMEM),
        out_shape=jax.ShapeDtypeStruct((1, 1), jnp.float32),
        scratch_shapes=[pltpu.SMEM((2,), jnp.float32)],
        compiler_params=pltpu.CompilerParams(
            dimension_semantics=("arbitrary",)),
    )(st, tt, stu_candidate_iou, tea_candidate_iou, sgt, tgt, Ms)
    return out[0, 0]


# speedup vs baseline: 20.5525x; 1.2668x over previous
import functools

import jax
import jax.numpy as jnp
from jax.experimental import pallas as pl
from jax.experimental.pallas import tpu as pltpu

N = 134400
C = 64
W = 16
RT = 8192
NB = (N + RT - 1) // RT


def _body(stu_ref, tea_ref, siou_ref, tiou_ref, sgt_ref, tgt_ref, ms_ref,
          out_ref, acc_ref):
    pid = pl.program_id(0)

    @pl.when(pid == 0)
    def _init():
        acc_ref[0] = 0.0
        acc_ref[1] = 0.0

    nvalid = N - pid * RT

    lane1 = jax.lax.broadcasted_iota(jnp.int32, (RT,), 0)
    m = jnp.logical_and(tiou_ref[...] >= siou_ref[...],
                        tgt_ref[...] == sgt_ref[...])
    m = jnp.logical_and(m, ms_ref[...])
    m = jnp.logical_and(m, lane1 < nvalid)
    mf = m.astype(jnp.float32)

    t = tea_ref[...]
    s = stu_ref[...]

    a8 = jax.lax.broadcasted_iota(jnp.int32, (8, C), 0) & 3
    c8 = jax.lax.broadcasted_iota(jnp.int32, (8, C), 1) // W
    p8 = (a8 == c8).astype(jnp.float32)
    aq = jax.lax.broadcasted_iota(jnp.int32, (C, 8), 1)
    cq = jax.lax.broadcasted_iota(jnp.int32, (C, 8), 0) // W
    q_mat = (aq == cq).astype(jnp.float32)

    et = jnp.exp(t)
    es = jnp.exp(s)
    bt8 = jax.lax.dot(p8, et)
    bs8 = jax.lax.dot(p8, es)
    btinv = jax.lax.dot(q_mat, 1.0 / bt8)

    elem1 = et * btinv * (t - s)
    l4 = jnp.log(bt8[0:4, :] / bs8[0:4, :])

    row_kl = jnp.sum(elem1, axis=0) - jnp.sum(l4, axis=0)
    row_kl = jnp.where(lane1 < nvalid, row_kl, 0.0)

    acc_ref[0] += jnp.sum(row_kl * mf)
    acc_ref[1] += jnp.sum(mf)

    @pl.when(pid == NB - 1)
    def _fin():
        out_ref[0, 0] = acc_ref[0] / (jnp.maximum(acc_ref[1], 1.0) * C)


@functools.partial(jax.jit, static_argnames=())
def kernel(stu_distri, tea_distri, stu_candidate_iou, tea_candidate_iou,
           stu_target_gt_idx, tea_target_gt_idx, Ms):
    st = stu_distri.T
    tt = tea_distri.T
    sgt = stu_target_gt_idx
    tgt = tea_target_gt_idx
    if sgt.dtype != jnp.int32:
        sgt = sgt.astype(jnp.int32)
        tgt = tgt.astype(jnp.int32)

    row_spec = pl.BlockSpec((C, RT), lambda i: (0, i))
    vec_spec = pl.BlockSpec((RT,), lambda i: (i,))

    out = pl.pallas_call(
        _body,
        grid=(NB,),
        in_specs=[row_spec, row_spec] + [vec_spec] * 5,
        out_specs=pl.BlockSpec(memory_space=pltpu.SMEM),
        out_shape=jax.ShapeDtypeStruct((1, 1), jnp.float32),
        scratch_shapes=[pltpu.SMEM((2,), jnp.float32)],
        compiler_params=pltpu.CompilerParams(
            dimension_semantics=("arbitrary",)),
    )(st, tt, stu_candidate_iou, tea_candidate_iou, sgt, tgt, Ms)
    return out[0, 0]


# speedup vs baseline: 22.2397x; 1.0821x over previous
import functools

import jax
import jax.numpy as jnp
from jax.experimental import pallas as pl
from jax.experimental.pallas import tpu as pltpu

N = 134400
C = 64
W = 16
RT = 16384
NB = (N + RT - 1) // RT


def _body(stu_ref, tea_ref, siou_ref, tiou_ref, sgt_ref, tgt_ref, ms_ref,
          out_ref, acc_ref):
    pid = pl.program_id(0)

    @pl.when(pid == 0)
    def _init():
        acc_ref[0] = 0.0
        acc_ref[1] = 0.0

    nvalid = N - pid * RT

    lane1 = jax.lax.broadcasted_iota(jnp.int32, (RT,), 0)
    m = jnp.logical_and(tiou_ref[...] >= siou_ref[...],
                        tgt_ref[...] == sgt_ref[...])
    m = jnp.logical_and(m, ms_ref[...])
    m = jnp.logical_and(m, lane1 < nvalid)
    mf = m.astype(jnp.float32)

    t = tea_ref[...]
    s = stu_ref[...]

    a8 = jax.lax.broadcasted_iota(jnp.int32, (8, C), 0) & 3
    c8 = jax.lax.broadcasted_iota(jnp.int32, (8, C), 1) // W
    p8 = (a8 == c8).astype(jnp.float32)
    aq = jax.lax.broadcasted_iota(jnp.int32, (C, 8), 1)
    cq = jax.lax.broadcasted_iota(jnp.int32, (C, 8), 0) // W
    q_mat = (aq == cq).astype(jnp.float32)

    et = jnp.exp(t)
    es = jnp.exp(s)
    bt8 = jax.lax.dot(p8, et)
    bs8 = jax.lax.dot(p8, es)
    btinv = jax.lax.dot(q_mat, 1.0 / bt8)

    elem1 = et * btinv * (t - s)
    l4 = jnp.log(bt8[0:4, :] / bs8[0:4, :])

    row_kl = jnp.sum(elem1, axis=0) - jnp.sum(l4, axis=0)
    row_kl = jnp.where(lane1 < nvalid, row_kl, 0.0)

    acc_ref[0] += jnp.sum(row_kl * mf)
    acc_ref[1] += jnp.sum(mf)

    @pl.when(pid == NB - 1)
    def _fin():
        out_ref[0, 0] = acc_ref[0] / (jnp.maximum(acc_ref[1], 1.0) * C)


@functools.partial(jax.jit, static_argnames=())
def kernel(stu_distri, tea_distri, stu_candidate_iou, tea_candidate_iou,
           stu_target_gt_idx, tea_target_gt_idx, Ms):
    st = stu_distri.T
    tt = tea_distri.T
    sgt = stu_target_gt_idx
    tgt = tea_target_gt_idx
    if sgt.dtype != jnp.int32:
        sgt = sgt.astype(jnp.int32)
        tgt = tgt.astype(jnp.int32)

    row_spec = pl.BlockSpec((C, RT), lambda i: (0, i))
    vec_spec = pl.BlockSpec((RT,), lambda i: (i,))

    out = pl.pallas_call(
        _body,
        grid=(NB,),
        in_specs=[row_spec, row_spec] + [vec_spec] * 5,
        out_specs=pl.BlockSpec(memory_space=pltpu.SMEM),
        out_shape=jax.ShapeDtypeStruct((1, 1), jnp.float32),
        scratch_shapes=[pltpu.SMEM((2,), jnp.float32)],
        compiler_params=pltpu.CompilerParams(
            dimension_semantics=("arbitrary",)),
    )(st, tt, stu_candidate_iou, tea_candidate_iou, sgt, tgt, Ms)
    return out[0, 0]


# speedup vs baseline: 23.6963x; 1.0655x over previous
import functools

import jax
import jax.numpy as jnp
from jax.experimental import pallas as pl
from jax.experimental.pallas import tpu as pltpu

N = 134400
C = 64
W = 16
RT = 16384
NB = (N + RT - 1) // RT


def _body(stu_ref, tea_ref, siou_ref, tiou_ref, sgt_ref, tgt_ref, ms_ref,
          out_ref, acc_ref):
    pid = pl.program_id(0)

    @pl.when(pid == 0)
    def _init():
        acc_ref[0] = 0.0
        acc_ref[1] = 0.0

    nvalid = N - pid * RT

    lane1 = jax.lax.broadcasted_iota(jnp.int32, (RT,), 0)
    m = jnp.logical_and(tiou_ref[...] >= siou_ref[...],
                        tgt_ref[...] == sgt_ref[...])
    m = jnp.logical_and(m, ms_ref[...])
    m = jnp.logical_and(m, lane1 < nvalid)
    mf = m.astype(jnp.float32)

    t = tea_ref[...]
    s = stu_ref[...]

    a8 = jax.lax.broadcasted_iota(jnp.int32, (8, C), 0) & 3
    c8 = jax.lax.broadcasted_iota(jnp.int32, (8, C), 1) // W
    p8 = (a8 == c8).astype(jnp.float32)

    et = jnp.exp(t)
    es = jnp.exp(s)
    z = et * (t - s)
    bt8 = jax.lax.dot(p8, et)
    bs8 = jax.lax.dot(p8, es)
    zg8 = jax.lax.dot(p8, z)

    l4 = jnp.log(bt8[0:4, :] / bs8[0:4, :])
    g4 = zg8[0:4, :] / bt8[0:4, :]

    row_kl = jnp.sum(g4 - l4, axis=0)
    row_kl = jnp.where(lane1 < nvalid, row_kl, 0.0)

    acc_ref[0] += jnp.sum(row_kl * mf)
    acc_ref[1] += jnp.sum(mf)

    @pl.when(pid == NB - 1)
    def _fin():
        out_ref[0, 0] = acc_ref[0] / (jnp.maximum(acc_ref[1], 1.0) * C)


@functools.partial(jax.jit, static_argnames=())
def kernel(stu_distri, tea_distri, stu_candidate_iou, tea_candidate_iou,
           stu_target_gt_idx, tea_target_gt_idx, Ms):
    st = stu_distri.T
    tt = tea_distri.T
    sgt = stu_target_gt_idx
    tgt = tea_target_gt_idx
    if sgt.dtype != jnp.int32:
        sgt = sgt.astype(jnp.int32)
        tgt = tgt.astype(jnp.int32)

    row_spec = pl.BlockSpec((C, RT), lambda i: (0, i))
    vec_spec = pl.BlockSpec((RT,), lambda i: (i,))

    out = pl.pallas_call(
        _body,
        grid=(NB,),
        in_specs=[row_spec, row_spec] + [vec_spec] * 5,
        out_specs=pl.BlockSpec(memory_space=pltpu.SMEM),
        out_shape=jax.ShapeDtypeStruct((1, 1), jnp.float32),
        scratch_shapes=[pltpu.SMEM((2,), jnp.float32)],
        compiler_params=pltpu.CompilerParams(
            dimension_semantics=("arbitrary",)),
    )(st, tt, stu_candidate_iou, tea_candidate_iou, sgt, tgt, Ms)
    return out[0, 0]


# speedup vs baseline: 24.6306x; 1.0394x over previous
import functools

import jax
import jax.numpy as jnp
from jax.experimental import pallas as pl
from jax.experimental.pallas import tpu as pltpu

N = 134400
C = 64
W = 16
RT = 16384
NB = (N + RT - 1) // RT


def _body(stu_ref, tea_ref, siou_ref, tiou_ref, sgt_ref, tgt_ref, ms_ref,
          out_ref, acc_ref):
    pid = pl.program_id(0)

    @pl.when(pid == 0)
    def _init():
        acc_ref[0] = 0.0
        acc_ref[1] = 0.0

    nvalid = N - pid * RT

    lane1 = jax.lax.broadcasted_iota(jnp.int32, (RT,), 0)
    m = jnp.logical_and(tiou_ref[...] >= siou_ref[...],
                        tgt_ref[...] == sgt_ref[...])
    m = jnp.logical_and(m, ms_ref[...])
    m = jnp.logical_and(m, lane1 < nvalid)
    mf = m.astype(jnp.float32)

    t = tea_ref[...]
    s = stu_ref[...]

    a8 = jax.lax.broadcasted_iota(jnp.int32, (8, C), 0) & 3
    c8 = jax.lax.broadcasted_iota(jnp.int32, (8, C), 1) // W
    p8 = (a8 == c8).astype(jnp.bfloat16)

    tb = t.astype(jnp.bfloat16)
    sb = s.astype(jnp.bfloat16)
    et = jnp.exp(tb)
    es = jnp.exp(sb)
    z = et * (tb - sb)
    f32 = jnp.float32
    bt8 = jax.lax.dot(p8, et, preferred_element_type=f32)
    bs8 = jax.lax.dot(p8, es, preferred_element_type=f32)
    zg8 = jax.lax.dot(p8, z, preferred_element_type=f32)

    l4 = jnp.log(bt8[0:4, :] / bs8[0:4, :])
    g4 = zg8[0:4, :] / bt8[0:4, :]

    row_kl = jnp.sum(g4 - l4, axis=0)
    row_kl = jnp.where(lane1 < nvalid, row_kl, 0.0)

    acc_ref[0] += jnp.sum(row_kl * mf)
    acc_ref[1] += jnp.sum(mf)

    @pl.when(pid == NB - 1)
    def _fin():
        out_ref[0, 0] = acc_ref[0] / (jnp.maximum(acc_ref[1], 1.0) * C)


@functools.partial(jax.jit, static_argnames=())
def kernel(stu_distri, tea_distri, stu_candidate_iou, tea_candidate_iou,
           stu_target_gt_idx, tea_target_gt_idx, Ms):
    st = stu_distri.T
    tt = tea_distri.T
    sgt = stu_target_gt_idx
    tgt = tea_target_gt_idx
    if sgt.dtype != jnp.int32:
        sgt = sgt.astype(jnp.int32)
        tgt = tgt.astype(jnp.int32)

    row_spec = pl.BlockSpec((C, RT), lambda i: (0, i))
    vec_spec = pl.BlockSpec((RT,), lambda i: (i,))

    out = pl.pallas_call(
        _body,
        grid=(NB,),
        in_specs=[row_spec, row_spec] + [vec_spec] * 5,
        out_specs=pl.BlockSpec(memory_space=pltpu.SMEM),
        out_shape=jax.ShapeDtypeStruct((1, 1), jnp.float32),
        scratch_shapes=[pltpu.SMEM((2,), jnp.float32)],
        compiler_params=pltpu.CompilerParams(
            dimension_semantics=("arbitrary",)),
    )(st, tt, stu_candidate_iou, tea_candidate_iou, sgt, tgt, Ms)
    return out[0, 0]
